# baseline (speedup 1.0000x reference)
import functools

import jax
import jax.numpy as jnp
from jax import lax
from jax.experimental import pallas as pl
from jax.experimental.pallas import tpu as pltpu
from jax.experimental.pallas import tpu_sc as plsc

N = 10000
E = 320000
F_IN = 160
H = 2
HID = 256

ROW_BLK = 2000
N_BLKS = N // ROW_BLK

LROW = 20928
EL = E + N
CHUNK = 64
CPT = 324
EPT = CPT * CHUNK
EL_PAD = 16 * EPT
NSC = 5000
NACC = 5120
RPT = NACC // 16
FE = HID + 16


def _lrelu(v):
    return jnp.where(v >= 0, v, 0.2 * v)


def _front_body(x_ref, w_ref, att_ref, xw0_ref, xw1_ref, a_ref):
    xw = jnp.dot(x_ref[:], w_ref[:], preferred_element_type=jnp.float32)
    one = jnp.ones((ROW_BLK, 1), jnp.float32)
    zpad = jnp.zeros((ROW_BLK, FE - HID - 1), jnp.float32)
    xw0_ref[:] = jnp.concatenate([xw[:, :HID], one, zpad], axis=1)
    xw1_ref[:] = jnp.concatenate([xw[:, HID:], one, zpad], axis=1)
    a_ref[:] = jnp.dot(xw, att_ref[:], preferred_element_type=jnp.float32)


def _front(x, W, att_mat):
    return pl.pallas_call(
        _front_body,
        grid=(N_BLKS,),
        in_specs=[
            pl.BlockSpec((ROW_BLK, F_IN), lambda i: (i, 0)),
            pl.BlockSpec((F_IN, H * HID), lambda i: (0, 0)),
            pl.BlockSpec((H * HID, 4), lambda i: (0, 0)),
        ],
        out_specs=[
            pl.BlockSpec((ROW_BLK, FE), lambda i: (i, 0)),
            pl.BlockSpec((ROW_BLK, FE), lambda i: (i, 0)),
            pl.BlockSpec((ROW_BLK, 4), lambda i: (i, 0)),
        ],
        out_shape=[
            jax.ShapeDtypeStruct((N, FE), jnp.float32),
            jax.ShapeDtypeStruct((N, FE), jnp.float32),
            jax.ShapeDtypeStruct((N, 4), jnp.float32),
        ],
    )(x, W, att_mat)


def _edge_sc(srcp, dstp, asrep, adrep, xwef):
    mesh = plsc.VectorSubcoreMesh(core_axis_name="c", subcore_axis_name="s")

    @functools.partial(
        pl.kernel, mesh=mesh,
        compiler_params=pltpu.CompilerParams(needs_layout_passes=False,
                                             use_tc_tiling_on_sc=False),
        out_type=jax.ShapeDtypeStruct((4 * NACC, FE), jnp.float32),
        scratch_types=[
            [pltpu.VMEM((CHUNK,), jnp.int32)] * 2,
            [pltpu.VMEM((CHUNK,), jnp.int32)] * 2,
            [pltpu.VMEM((CHUNK,), jnp.int32)] * 2,
            [pltpu.VMEM((CHUNK,), jnp.int32)] * 2,
            [pltpu.VMEM((CHUNK,), jnp.float32)] * 2,
            [pltpu.VMEM((CHUNK, 16), jnp.float32)] * 2,
            [pltpu.VMEM((CHUNK, 16), jnp.float32)] * 2,
            [pltpu.VMEM((CHUNK, FE), jnp.float32)] * 2,
            pltpu.VMEM((80,), jnp.int32),
            pltpu.HBM((2 * 16 * LROW,), jnp.int32),
            pltpu.VMEM_SHARED((NACC, FE), jnp.float32),
            [pltpu.SemaphoreType.DMA] * 2,
            [pltpu.SemaphoreType.DMA] * 2,
            [pltpu.SemaphoreType.DMA] * 2,
        ],
    )
    def body(srcp_h, dstp_h, asr_h, adr_h, xwe_h, out,
             srcbuf, dstbuf, sidxbuf, locbuf, okbuf, sabuf, dabuf, staging,
             chunkbuf, lists, acc, sem, sem2, sem3):
        sc = lax.axis_index("c")
        s = lax.axis_index("s")
        tile_base = s * EPT
        lo = sc * NSC
        lbase = (sc * 16 + s) * LROW

        z16 = jnp.zeros((16,), jnp.float32)
        zi16 = jnp.zeros((16,), jnp.int32)
        i16 = lax.iota(jnp.int32, 16)
        SENT = NACC - 1

        def prep_body(k, cnt8):
            off = tile_base + k * CHUNK
            pltpu.sync_copy(srcp_h.at[pl.ds(off, CHUNK)], srcbuf[0])
            pltpu.sync_copy(dstp_h.at[pl.ds(off, CHUNK)], dstbuf[0])
            lcnt = jnp.int32(0)
            for g in range(CHUNK // 16):
                s16 = srcbuf[0][pl.ds(16 * g, 16)]
                d16 = dstbuf[0][pl.ds(16 * g, 16)]
                eid = off + 16 * g + i16
                m = (eid < EL) & (d16 >= lo) & (d16 < lo + NSC)
                mi = jnp.where(m, 1, 0).astype(jnp.int32)
                pos = lcnt + plsc.cumsum(mi) - mi
                plsc.store_scatter(chunkbuf, [pos],
                                   (s16 << 13) | (d16 - lo), mask=m)
                lcnt = lcnt + jnp.sum(mi)
            for g in range(5):
                idx = 16 * g + i16
                cur = chunkbuf[pl.ds(16 * g, 16)]
                chunkbuf[pl.ds(16 * g, 16)] = jnp.where(idx < lcnt, cur,
                                                        zi16 + SENT)
            pltpu.sync_copy(chunkbuf.at[pl.ds(0, 72)],
                            lists.at[pl.ds(pl.multiple_of(lbase + cnt8, 8), 72)])
            return cnt8 + (((lcnt + 7) >> 3) << 3)
        cnt8 = lax.fori_loop(0, CPT, prep_body, jnp.int32(0))
        for g in range(5):
            chunkbuf[pl.ds(16 * g, 16)] = zi16 + SENT
        pltpu.sync_copy(chunkbuf.at[pl.ds(0, 64)],
                        lists.at[pl.ds(pl.multiple_of(lbase + cnt8, 8), 64)])
        pltpu.sync_copy(chunkbuf.at[pl.ds(0, 64)],
                        lists.at[pl.ds(pl.multiple_of(lbase + cnt8 + 64, 8), 64)])
        nchunks = (cnt8 + CHUNK - 1) // CHUNK

        def head_body(h, carry0):
            def zrow(r, carry):
                for c in range(FE // 16):
                    staging[0][r, pl.ds(16 * c, 16)] = z16
                return carry
            lax.fori_loop(0, CHUNK, zrow, 0)
            offs = s * RPT
            def zacc(z, carry):
                pltpu.sync_copy(staging[0],
                                acc.at[pl.ds(offs + z * CHUNK, CHUNK)])
                return carry
            lax.fori_loop(0, RPT // CHUNK, zacc, 0)
            plsc.subcore_barrier()

            hN = h * N

            def stage_issue(k, b):
                pltpu.sync_copy(
                    lists.at[pl.ds(pl.multiple_of(lbase + k * CHUNK, 8), CHUNK)],
                    srcbuf[b])
                for g in range(CHUNK // 16):
                    v16 = srcbuf[b][pl.ds(16 * g, 16)]
                    loc16 = v16 & (2 ** 13 - 1)
                    s16 = v16 >> 13
                    ok = loc16 != SENT
                    okbuf[b][pl.ds(16 * g, 16)] = jnp.where(ok, 1.0, 0.0)
                    locbuf[b][pl.ds(16 * g, 16)] = loc16
                    sidxbuf[b][pl.ds(16 * g, 16)] = s16 + hN
                    dstbuf[b][pl.ds(16 * g, 16)] = loc16 + lo + hN
                pltpu.async_copy(xwe_h.at[sidxbuf[b]], staging[b], sem[b])
                pltpu.async_copy(asr_h.at[sidxbuf[b]], sabuf[b], sem2[b])
                pltpu.async_copy(adr_h.at[dstbuf[b]], dabuf[b], sem3[b])

            def wait_gathers(b):
                pltpu.make_async_copy(xwe_h.at[sidxbuf[b]], staging[b], sem[b]).wait()
                pltpu.make_async_copy(asr_h.at[sidxbuf[b]], sabuf[b], sem2[b]).wait()
                pltpu.make_async_copy(adr_h.at[dstbuf[b]], dabuf[b], sem3[b]).wait()

            def scale_scatter(b):
                def srow(r, carry2):
                    lg = sabuf[b][r, pl.ds(0, 16)] + dabuf[b][r, pl.ds(0, 16)]
                    lg = jnp.where(lg >= 0, lg, 0.2 * lg)
                    okspl = plsc.load_gather(
                        okbuf[b], [jnp.zeros((16,), jnp.int32) + r])
                    wspl = jnp.exp(lg) * okspl
                    for c in range(FE // 16):
                        staging[b][r, pl.ds(16 * c, 16)] = (
                            staging[b][r, pl.ds(16 * c, 16)] * wspl)
                    return carry2
                lax.fori_loop(0, CHUNK, srow, 0)
                pltpu.sync_copy(staging[b], acc.at[locbuf[b]], add=True)

            stage_issue(0, 0)

            def pair_body(j, carry):
                k0 = 2 * j
                stage_issue(jnp.minimum(k0 + 1, nchunks), 1)
                wait_gathers(0)
                scale_scatter(0)
                stage_issue(jnp.minimum(k0 + 2, nchunks), 0)
                wait_gathers(1)
                scale_scatter(1)
                return carry
            lax.fori_loop(0, (nchunks + 1) // 2, pair_body, 0)
            wait_gathers(0)
            plsc.subcore_barrier()

            obase = h * 2 * NACC + sc * NACC + offs
            def cpout(z, carry):
                pltpu.sync_copy(acc.at[pl.ds(offs + z * CHUNK, CHUNK)],
                                out.at[pl.ds(obase + z * CHUNK, CHUNK)])
                return carry
            lax.fori_loop(0, RPT // CHUNK, cpout, 0)
            plsc.subcore_barrier()
            return carry0
        lax.fori_loop(0, 2, head_body, 0)

    return body(srcp, dstp, asrep, adrep, xwef)


def _mid_body(s0_ref, s1_ref, den_ref, bias_ref, h_ref, ps_ref, pq_ref):
    den0 = den_ref[:, 0:1]
    den1 = den_ref[:, 1:2]
    h = (s0_ref[:] / (den0 + 1e-16) + s1_ref[:] / (den1 + 1e-16)) * 0.5
    h = h + bias_ref[:]
    h_ref[:] = h
    ps_ref[0, 0, :] = jnp.sum(h, axis=0)
    pq_ref[0, 0, :] = jnp.sum(h * h, axis=0)


def _mid(s0, s1, den, bias):
    return pl.pallas_call(
        _mid_body,
        grid=(N_BLKS,),
        in_specs=[
            pl.BlockSpec((ROW_BLK, HID), lambda i: (i, 0)),
            pl.BlockSpec((ROW_BLK, HID), lambda i: (i, 0)),
            pl.BlockSpec((ROW_BLK, 2), lambda i: (i, 0)),
            pl.BlockSpec((1, HID), lambda i: (0, 0)),
        ],
        out_specs=[
            pl.BlockSpec((ROW_BLK, HID), lambda i: (i, 0)),
            pl.BlockSpec((1, 1, HID), lambda i: (i, 0, 0)),
            pl.BlockSpec((1, 1, HID), lambda i: (i, 0, 0)),
        ],
        out_shape=[
            jax.ShapeDtypeStruct((N, HID), jnp.float32),
            jax.ShapeDtypeStruct((N_BLKS, 1, HID), jnp.float32),
            jax.ShapeDtypeStruct((N_BLKS, 1, HID), jnp.float32),
        ],
    )(s0, s1, den, bias)


def _mlp_body(h_ref, sc_ref, sh_ref, w1_ref, b1_ref, w2_ref, b2_ref,
              w3_ref, b3_ref, w4_ref, b4_ref, o_ref):
    h = _lrelu(h_ref[:] * sc_ref[:] + sh_ref[:])
    h = _lrelu(jnp.dot(h, w1_ref[:], preferred_element_type=jnp.float32) + b1_ref[:])
    h = _lrelu(jnp.dot(h, w2_ref[:], preferred_element_type=jnp.float32) + b2_ref[:])
    h = _lrelu(jnp.dot(h, w3_ref[:], preferred_element_type=jnp.float32) + b3_ref[:])
    h = _lrelu(jnp.dot(h, w4_ref[:], preferred_element_type=jnp.float32) + b4_ref[:])
    o_ref[:] = h


def _mlp(h, scale, shift, w1, b1, w2, b2, w3, b3, w4, b4):
    full = lambda shape: pl.BlockSpec(shape, lambda i: tuple(0 for _ in shape))
    return pl.pallas_call(
        _mlp_body,
        grid=(N_BLKS,),
        in_specs=[
            pl.BlockSpec((ROW_BLK, HID), lambda i: (i, 0)),
            full((1, HID)), full((1, HID)),
            full((HID, HID // 2)), full((1, HID // 2)),
            full((HID // 2, HID // 4)), full((1, HID // 4)),
            full((HID // 4, HID // 4)), full((1, HID // 4)),
            full((HID // 4, 128)), full((1, 128)),
        ],
        out_specs=pl.BlockSpec((ROW_BLK, 128), lambda i: (i, 0)),
        out_shape=jax.ShapeDtypeStruct((N, 128), jnp.float32),
    )(h, scale, shift, w1, b1, w2, b2, w3, b3, w4, b4)


def kernel(normal_features, noise, edge_index, batch, W, att_src, att_dst,
           conv_bias, bn_gamma, bn_beta, w1, b1, w2, b2, w3, b3, w4, b4):
    x = jnp.concatenate([normal_features, noise], axis=1)
    loop = jnp.arange(N, dtype=edge_index.dtype)
    src = jnp.concatenate([edge_index[0], loop])
    dst = jnp.concatenate([edge_index[1], loop])

    z = jnp.zeros((HID,), jnp.float32)
    att_mat = jnp.stack([
        jnp.concatenate([att_src[0], z]),
        jnp.concatenate([z, att_src[1]]),
        jnp.concatenate([att_dst[0], z]),
        jnp.concatenate([z, att_dst[1]]),
    ], axis=1)

    xw0e, xw1e, a = _front(x, W, att_mat)

    m = _lrelu(jnp.max(a[:, 0:2], axis=0) + jnp.max(a[:, 2:4], axis=0))
    xwef = jnp.concatenate([xw0e * jnp.exp(-m[0]), xw1e * jnp.exp(-m[1])])

    pad = jnp.zeros((EL_PAD - EL,), dtype=edge_index.dtype)
    srcp = jnp.concatenate([src, pad])
    dstp = jnp.concatenate([dst, pad])

    asrep = jnp.broadcast_to(a[:, 0:2].T.reshape(2 * N, 1), (2 * N, 16))
    adrep = jnp.broadcast_to(a[:, 2:4].T.reshape(2 * N, 1), (2 * N, 16))
    outf = _edge_sc(srcp, dstp, asrep, adrep, xwef)
    s0e = jnp.concatenate([outf[:NSC], outf[NACC:NACC + NSC]], axis=0)
    s1e = jnp.concatenate([outf[2 * NACC:2 * NACC + NSC],
                           outf[3 * NACC:3 * NACC + NSC]], axis=0)
    s0 = s0e[:, :HID]
    s1 = s1e[:, :HID]
    den = jnp.stack([s0e[:, HID], s1e[:, HID]], axis=1)

    h, ps, pq = _mid(s0, s1, den, conv_bias.reshape(1, HID))
    mu = ps.sum(axis=(0, 1)) / N
    var = pq.sum(axis=(0, 1)) / N - mu * mu
    scale = bn_gamma / jnp.sqrt(var + 1e-5)
    shift = bn_beta - mu * scale

    return _mlp(h, scale.reshape(1, HID), shift.reshape(1, HID),
                w1, b1.reshape(1, -1), w2, b2.reshape(1, -1),
                w3, b3.reshape(1, -1), w4, b4.reshape(1, -1))


# speedup vs baseline: 2.5472x; 2.5472x over previous
import functools

import jax
import jax.numpy as jnp
from jax import lax
from jax.experimental import pallas as pl
from jax.experimental.pallas import tpu as pltpu
from jax.experimental.pallas import tpu_sc as plsc

N = 10000
E = 320000
F_IN = 160
H = 2
HID = 256

ROW_BLK = 2000
N_BLKS = N // ROW_BLK

LROW = 20928
EL = E + N
CHUNK = 64
CPT = 324
EPT = CPT * CHUNK
EL_PAD = 16 * EPT
NSC = 5000
NACC = 5120
RPT = NACC // 16
FE = HID + 16


def _lrelu(v):
    return jnp.where(v >= 0, v, 0.2 * v)


def _front_body(x_ref, w_ref, att_ref, xw0_ref, xw1_ref, a_ref):
    xw = jnp.dot(x_ref[:], w_ref[:], preferred_element_type=jnp.float32)
    one = jnp.ones((ROW_BLK, 1), jnp.float32)
    zpad = jnp.zeros((ROW_BLK, FE - HID - 1), jnp.float32)
    xw0_ref[:] = jnp.concatenate([xw[:, :HID], one, zpad], axis=1)
    xw1_ref[:] = jnp.concatenate([xw[:, HID:], one, zpad], axis=1)
    a_ref[:] = jnp.dot(xw, att_ref[:], preferred_element_type=jnp.float32)


def _front(x, W, att_mat):
    return pl.pallas_call(
        _front_body,
        grid=(N_BLKS,),
        in_specs=[
            pl.BlockSpec((ROW_BLK, F_IN), lambda i: (i, 0)),
            pl.BlockSpec((F_IN, H * HID), lambda i: (0, 0)),
            pl.BlockSpec((H * HID, 4), lambda i: (0, 0)),
        ],
        out_specs=[
            pl.BlockSpec((ROW_BLK, FE), lambda i: (i, 0)),
            pl.BlockSpec((ROW_BLK, FE), lambda i: (i, 0)),
            pl.BlockSpec((ROW_BLK, 4), lambda i: (i, 0)),
        ],
        out_shape=[
            jax.ShapeDtypeStruct((N, FE), jnp.float32),
            jax.ShapeDtypeStruct((N, FE), jnp.float32),
            jax.ShapeDtypeStruct((N, 4), jnp.float32),
        ],
    )(x, W, att_mat)


def _edge_sc(srcp, dstp, asrep, adrep, xwef):
    mesh = plsc.VectorSubcoreMesh(core_axis_name="c", subcore_axis_name="s")

    @functools.partial(
        pl.kernel, mesh=mesh,
        compiler_params=pltpu.CompilerParams(needs_layout_passes=False,
                                             use_tc_tiling_on_sc=False),
        out_type=jax.ShapeDtypeStruct((4 * NACC, FE), jnp.float32),
        scratch_types=[
            [pltpu.VMEM((CHUNK,), jnp.int32)] * 2,
            [pltpu.VMEM((CHUNK,), jnp.int32)] * 2,
            [pltpu.VMEM((CHUNK,), jnp.int32)] * 2,
            [pltpu.VMEM((CHUNK,), jnp.int32)] * 2,
            [pltpu.VMEM((CHUNK,), jnp.float32)] * 2,
            [pltpu.VMEM((CHUNK, 16), jnp.float32)] * 2,
            [pltpu.VMEM((CHUNK, 16), jnp.float32)] * 2,
            [pltpu.VMEM((CHUNK, FE), jnp.float32)] * 2,
            pltpu.VMEM((592,), jnp.int32),
            pltpu.VMEM((576,), jnp.int32),
            pltpu.VMEM((576,), jnp.int32),
            pltpu.HBM((2 * 16 * LROW,), jnp.int32),
            pltpu.VMEM_SHARED((NACC, FE), jnp.float32),
            [pltpu.SemaphoreType.DMA] * 2,
            [pltpu.SemaphoreType.DMA] * 2,
            [pltpu.SemaphoreType.DMA] * 2,
        ],
    )
    def body(srcp_h, dstp_h, asr_h, adr_h, xwe_h, out,
             srcbuf, dstbuf, sidxbuf, locbuf, okbuf, sabuf, dabuf, staging,
             chunkbuf, pbuf_s, pbuf_d, lists, acc, sem, sem2, sem3):
        sc = lax.axis_index("c")
        s = lax.axis_index("s")
        tile_base = s * EPT
        lo = sc * NSC
        lbase = (sc * 16 + s) * LROW

        z16 = jnp.zeros((16,), jnp.float32)
        zi16 = jnp.zeros((16,), jnp.int32)
        i16 = lax.iota(jnp.int32, 16)
        SENT = NACC - 1

        PCH = 576
        def prep_body(k, cnt8):
            off = tile_base + k * PCH
            pltpu.sync_copy(srcp_h.at[pl.ds(off, PCH)], pbuf_s)
            pltpu.sync_copy(dstp_h.at[pl.ds(off, PCH)], pbuf_d)
            lcnt = jnp.int32(0)
            for g in range(PCH // 16):
                s16 = pbuf_s[pl.ds(16 * g, 16)]
                d16 = pbuf_d[pl.ds(16 * g, 16)]
                eid = off + 16 * g + i16
                m = (eid < EL) & (d16 >= lo) & (d16 < lo + NSC)
                mi = jnp.where(m, 1, 0).astype(jnp.int32)
                pos = lcnt + plsc.cumsum(mi) - mi
                plsc.store_scatter(chunkbuf, [pos],
                                   (s16 << 13) | (d16 - lo), mask=m)
                lcnt = lcnt + jnp.sum(mi)
            for g in range(37):
                idx = 16 * g + i16
                cur = chunkbuf[pl.ds(16 * g, 16)]
                chunkbuf[pl.ds(16 * g, 16)] = jnp.where(idx < lcnt, cur,
                                                        zi16 + SENT)
            pltpu.sync_copy(chunkbuf.at[pl.ds(0, 584)],
                            lists.at[pl.ds(pl.multiple_of(lbase + cnt8, 8), 584)])
            return cnt8 + (((lcnt + 7) >> 3) << 3)
        cnt8 = lax.fori_loop(0, EPT // PCH, prep_body, jnp.int32(0))
        for g in range(5):
            chunkbuf[pl.ds(16 * g, 16)] = zi16 + SENT
        pltpu.sync_copy(chunkbuf.at[pl.ds(0, 64)],
                        lists.at[pl.ds(pl.multiple_of(lbase + cnt8, 8), 64)])
        pltpu.sync_copy(chunkbuf.at[pl.ds(0, 64)],
                        lists.at[pl.ds(pl.multiple_of(lbase + cnt8 + 64, 8), 64)])
        nchunks = (cnt8 + CHUNK - 1) // CHUNK

        def head_body(h, carry0):
            def zrow(r, carry):
                for c in range(FE // 16):
                    staging[0][r, pl.ds(16 * c, 16)] = z16
                return carry
            lax.fori_loop(0, CHUNK, zrow, 0)
            offs = s * RPT
            def zacc(z, carry):
                pltpu.sync_copy(staging[0],
                                acc.at[pl.ds(offs + z * CHUNK, CHUNK)])
                return carry
            lax.fori_loop(0, RPT // CHUNK, zacc, 0)
            plsc.subcore_barrier()

            hN = h * N

            def stage_issue(k, b):
                pltpu.sync_copy(
                    lists.at[pl.ds(pl.multiple_of(lbase + k * CHUNK, 8), CHUNK)],
                    srcbuf[b])
                for g in range(CHUNK // 16):
                    v16 = srcbuf[b][pl.ds(16 * g, 16)]
                    loc16 = v16 & (2 ** 13 - 1)
                    s16 = v16 >> 13
                    ok = loc16 != SENT
                    okbuf[b][pl.ds(16 * g, 16)] = jnp.where(ok, 1.0, 0.0)
                    locbuf[b][pl.ds(16 * g, 16)] = loc16
                    sidxbuf[b][pl.ds(16 * g, 16)] = s16 + hN
                    dstbuf[b][pl.ds(16 * g, 16)] = loc16 + lo + hN
                pltpu.async_copy(xwe_h.at[sidxbuf[b]], staging[b], sem[b])
                pltpu.async_copy(asr_h.at[sidxbuf[b]], sabuf[b], sem2[b])
                pltpu.async_copy(adr_h.at[dstbuf[b]], dabuf[b], sem3[b])

            def wait_gathers(b):
                pltpu.make_async_copy(xwe_h.at[sidxbuf[b]], staging[b], sem[b]).wait()
                pltpu.make_async_copy(asr_h.at[sidxbuf[b]], sabuf[b], sem2[b]).wait()
                pltpu.make_async_copy(adr_h.at[dstbuf[b]], dabuf[b], sem3[b]).wait()

            def scale_scatter(b):
                def srow(r, carry2):
                    lg = sabuf[b][r, pl.ds(0, 16)] + dabuf[b][r, pl.ds(0, 16)]
                    lg = jnp.where(lg >= 0, lg, 0.2 * lg)
                    okspl = plsc.load_gather(
                        okbuf[b], [jnp.zeros((16,), jnp.int32) + r])
                    wspl = jnp.exp(lg) * okspl
                    for c in range(FE // 16):
                        staging[b][r, pl.ds(16 * c, 16)] = (
                            staging[b][r, pl.ds(16 * c, 16)] * wspl)
                    return carry2
                lax.fori_loop(0, CHUNK, srow, 0)
                pltpu.sync_copy(staging[b], acc.at[locbuf[b]], add=True)

            stage_issue(0, 0)

            def pair_body(j, carry):
                k0 = 2 * j
                stage_issue(jnp.minimum(k0 + 1, nchunks), 1)
                wait_gathers(0)
                scale_scatter(0)
                stage_issue(jnp.minimum(k0 + 2, nchunks), 0)
                wait_gathers(1)
                scale_scatter(1)
                return carry
            lax.fori_loop(0, (nchunks + 1) // 2, pair_body, 0)
            wait_gathers(0)
            plsc.subcore_barrier()

            obase = h * 2 * NACC + sc * NACC + offs
            def cpout(z, carry):
                pltpu.sync_copy(acc.at[pl.ds(offs + z * CHUNK, CHUNK)],
                                out.at[pl.ds(obase + z * CHUNK, CHUNK)])
                return carry
            lax.fori_loop(0, RPT // CHUNK, cpout, 0)
            plsc.subcore_barrier()
            return carry0
        lax.fori_loop(0, 2, head_body, 0)

    return body(srcp, dstp, asrep, adrep, xwef)


def _mid_body(s0_ref, s1_ref, den_ref, bias_ref, h_ref, ps_ref, pq_ref):
    den0 = den_ref[:, 0:1]
    den1 = den_ref[:, 1:2]
    h = (s0_ref[:] / (den0 + 1e-16) + s1_ref[:] / (den1 + 1e-16)) * 0.5
    h = h + bias_ref[:]
    h_ref[:] = h
    ps_ref[0, 0, :] = jnp.sum(h, axis=0)
    pq_ref[0, 0, :] = jnp.sum(h * h, axis=0)


def _mid(s0, s1, den, bias):
    return pl.pallas_call(
        _mid_body,
        grid=(N_BLKS,),
        in_specs=[
            pl.BlockSpec((ROW_BLK, HID), lambda i: (i, 0)),
            pl.BlockSpec((ROW_BLK, HID), lambda i: (i, 0)),
            pl.BlockSpec((ROW_BLK, 2), lambda i: (i, 0)),
            pl.BlockSpec((1, HID), lambda i: (0, 0)),
        ],
        out_specs=[
            pl.BlockSpec((ROW_BLK, HID), lambda i: (i, 0)),
            pl.BlockSpec((1, 1, HID), lambda i: (i, 0, 0)),
            pl.BlockSpec((1, 1, HID), lambda i: (i, 0, 0)),
        ],
        out_shape=[
            jax.ShapeDtypeStruct((N, HID), jnp.float32),
            jax.ShapeDtypeStruct((N_BLKS, 1, HID), jnp.float32),
            jax.ShapeDtypeStruct((N_BLKS, 1, HID), jnp.float32),
        ],
    )(s0, s1, den, bias)


def _mlp_body(h_ref, sc_ref, sh_ref, w1_ref, b1_ref, w2_ref, b2_ref,
              w3_ref, b3_ref, w4_ref, b4_ref, o_ref):
    h = _lrelu(h_ref[:] * sc_ref[:] + sh_ref[:])
    h = _lrelu(jnp.dot(h, w1_ref[:], preferred_element_type=jnp.float32) + b1_ref[:])
    h = _lrelu(jnp.dot(h, w2_ref[:], preferred_element_type=jnp.float32) + b2_ref[:])
    h = _lrelu(jnp.dot(h, w3_ref[:], preferred_element_type=jnp.float32) + b3_ref[:])
    h = _lrelu(jnp.dot(h, w4_ref[:], preferred_element_type=jnp.float32) + b4_ref[:])
    o_ref[:] = h


def _mlp(h, scale, shift, w1, b1, w2, b2, w3, b3, w4, b4):
    full = lambda shape: pl.BlockSpec(shape, lambda i: tuple(0 for _ in shape))
    return pl.pallas_call(
        _mlp_body,
        grid=(N_BLKS,),
        in_specs=[
            pl.BlockSpec((ROW_BLK, HID), lambda i: (i, 0)),
            full((1, HID)), full((1, HID)),
            full((HID, HID // 2)), full((1, HID // 2)),
            full((HID // 2, HID // 4)), full((1, HID // 4)),
            full((HID // 4, HID // 4)), full((1, HID // 4)),
            full((HID // 4, 128)), full((1, 128)),
        ],
        out_specs=pl.BlockSpec((ROW_BLK, 128), lambda i: (i, 0)),
        out_shape=jax.ShapeDtypeStruct((N, 128), jnp.float32),
    )(h, scale, shift, w1, b1, w2, b2, w3, b3, w4, b4)


def kernel(normal_features, noise, edge_index, batch, W, att_src, att_dst,
           conv_bias, bn_gamma, bn_beta, w1, b1, w2, b2, w3, b3, w4, b4):
    x = jnp.concatenate([normal_features, noise], axis=1)
    loop = jnp.arange(N, dtype=edge_index.dtype)
    src = jnp.concatenate([edge_index[0], loop])
    dst = jnp.concatenate([edge_index[1], loop])

    z = jnp.zeros((HID,), jnp.float32)
    att_mat = jnp.stack([
        jnp.concatenate([att_src[0], z]),
        jnp.concatenate([z, att_src[1]]),
        jnp.concatenate([att_dst[0], z]),
        jnp.concatenate([z, att_dst[1]]),
    ], axis=1)

    xw0e, xw1e, a = _front(x, W, att_mat)

    m = _lrelu(jnp.max(a[:, 0:2], axis=0) + jnp.max(a[:, 2:4], axis=0))
    xwef = jnp.concatenate([xw0e * jnp.exp(-m[0]), xw1e * jnp.exp(-m[1])])

    pad = jnp.zeros((EL_PAD - EL,), dtype=edge_index.dtype)
    srcp = jnp.concatenate([src, pad])
    dstp = jnp.concatenate([dst, pad])

    asrep = jnp.broadcast_to(a[:, 0:2].T.reshape(2 * N, 1), (2 * N, 16))
    adrep = jnp.broadcast_to(a[:, 2:4].T.reshape(2 * N, 1), (2 * N, 16))
    outf = _edge_sc(srcp, dstp, asrep, adrep, xwef)
    s0e = jnp.concatenate([outf[:NSC], outf[NACC:NACC + NSC]], axis=0)
    s1e = jnp.concatenate([outf[2 * NACC:2 * NACC + NSC],
                           outf[3 * NACC:3 * NACC + NSC]], axis=0)
    s0 = s0e[:, :HID]
    s1 = s1e[:, :HID]
    den = jnp.stack([s0e[:, HID], s1e[:, HID]], axis=1)

    h, ps, pq = _mid(s0, s1, den, conv_bias.reshape(1, HID))
    mu = ps.sum(axis=(0, 1)) / N
    var = pq.sum(axis=(0, 1)) / N - mu * mu
    scale = bn_gamma / jnp.sqrt(var + 1e-5)
    shift = bn_beta - mu * scale

    return _mlp(h, scale.reshape(1, HID), shift.reshape(1, HID),
                w1, b1.reshape(1, -1), w2, b2.reshape(1, -1),
                w3, b3.reshape(1, -1), w4, b4.reshape(1, -1))


# speedup vs baseline: 2.5659x; 1.0073x over previous
import functools

import jax
import jax.numpy as jnp
from jax import lax
from jax.experimental import pallas as pl
from jax.experimental.pallas import tpu as pltpu
from jax.experimental.pallas import tpu_sc as plsc

N = 10000
E = 320000
F_IN = 160
H = 2
HID = 256

ROW_BLK = 2000
N_BLKS = N // ROW_BLK

LROW = 20928
EL = E + N
CHUNK = 64
CPT = 324
EPT = CPT * CHUNK
EL_PAD = 16 * EPT
NSC = 5000
NACC = 5120
RPT = NACC // 16
FE = HID + 16


def _lrelu(v):
    return jnp.where(v >= 0, v, 0.2 * v)


def _front_body(x_ref, w_ref, att_ref, xw0_ref, xw1_ref, a_ref):
    xw = jnp.dot(x_ref[:], w_ref[:], preferred_element_type=jnp.float32)
    one = jnp.ones((ROW_BLK, 1), jnp.float32)
    zpad = jnp.zeros((ROW_BLK, FE - HID - 1), jnp.float32)
    xw0_ref[:] = jnp.concatenate([xw[:, :HID], one, zpad], axis=1)
    xw1_ref[:] = jnp.concatenate([xw[:, HID:], one, zpad], axis=1)
    a_ref[:] = jnp.dot(xw, att_ref[:], preferred_element_type=jnp.float32)


def _front(x, W, att_mat):
    return pl.pallas_call(
        _front_body,
        grid=(N_BLKS,),
        in_specs=[
            pl.BlockSpec((ROW_BLK, F_IN), lambda i: (i, 0)),
            pl.BlockSpec((F_IN, H * HID), lambda i: (0, 0)),
            pl.BlockSpec((H * HID, 4), lambda i: (0, 0)),
        ],
        out_specs=[
            pl.BlockSpec((ROW_BLK, FE), lambda i: (i, 0)),
            pl.BlockSpec((ROW_BLK, FE), lambda i: (i, 0)),
            pl.BlockSpec((ROW_BLK, 4), lambda i: (i, 0)),
        ],
        out_shape=[
            jax.ShapeDtypeStruct((N, FE), jnp.float32),
            jax.ShapeDtypeStruct((N, FE), jnp.float32),
            jax.ShapeDtypeStruct((N, 4), jnp.float32),
        ],
    )(x, W, att_mat)


def _edge_sc(srcp, dstp, asrep, adrep, xwef):
    mesh = plsc.VectorSubcoreMesh(core_axis_name="c", subcore_axis_name="s")

    @functools.partial(
        pl.kernel, mesh=mesh,
        compiler_params=pltpu.CompilerParams(needs_layout_passes=False,
                                             use_tc_tiling_on_sc=False),
        out_type=jax.ShapeDtypeStruct((4 * NACC, FE), jnp.float32),
        scratch_types=[
            [pltpu.VMEM((CHUNK,), jnp.int32)] * 2,
            [pltpu.VMEM((CHUNK,), jnp.int32)] * 2,
            [pltpu.VMEM((CHUNK,), jnp.int32)] * 2,
            [pltpu.VMEM((CHUNK,), jnp.int32)] * 2,
            [pltpu.VMEM((CHUNK,), jnp.float32)] * 2,
            [pltpu.VMEM((CHUNK, 16), jnp.float32)] * 2,
            [pltpu.VMEM((CHUNK, 16), jnp.float32)] * 2,
            [pltpu.VMEM((CHUNK, FE), jnp.float32)] * 2,
            pltpu.VMEM((592,), jnp.int32),
            pltpu.VMEM((576,), jnp.int32),
            pltpu.VMEM((576,), jnp.int32),
            pltpu.HBM((2 * 16 * LROW,), jnp.int32),
            pltpu.VMEM_SHARED((NACC, FE), jnp.float32),
            [pltpu.SemaphoreType.DMA] * 2,
            [pltpu.SemaphoreType.DMA] * 2,
            [pltpu.SemaphoreType.DMA] * 2,
        ],
    )
    def body(srcp_h, dstp_h, asr_h, adr_h, xwe_h, out,
             srcbuf, dstbuf, sidxbuf, locbuf, okbuf, sabuf, dabuf, staging,
             chunkbuf, pbuf_s, pbuf_d, lists, acc, sem, sem2, sem3):
        sc = lax.axis_index("c")
        s = lax.axis_index("s")
        tile_base = s * EPT
        lo = sc * NSC
        lbase = (sc * 16 + s) * LROW

        z16 = jnp.zeros((16,), jnp.float32)
        zi16 = jnp.zeros((16,), jnp.int32)
        i16 = lax.iota(jnp.int32, 16)
        SENT = NACC - 1

        PCH = 576
        def prep_body(k, cnt):
            off = tile_base + k * PCH
            pltpu.sync_copy(srcp_h.at[pl.ds(off, PCH)], pbuf_s)
            pltpu.sync_copy(dstp_h.at[pl.ds(off, PCH)], pbuf_d)
            base = cnt & 7
            lcnt = base
            for g in range(PCH // 16):
                s16 = pbuf_s[pl.ds(16 * g, 16)]
                d16 = pbuf_d[pl.ds(16 * g, 16)]
                eid = off + 16 * g + i16
                m = (eid < EL) & (d16 >= lo) & (d16 < lo + NSC)
                mi = jnp.where(m, 1, 0).astype(jnp.int32)
                pos = lcnt + plsc.cumsum(mi) - mi
                plsc.store_scatter(chunkbuf, [pos],
                                   (s16 << 13) | (d16 - lo), mask=m)
                lcnt = lcnt + jnp.sum(mi)
            for g in range(37):
                idx = 16 * g + i16
                cur = chunkbuf[pl.ds(16 * g, 16)]
                chunkbuf[pl.ds(16 * g, 16)] = jnp.where(idx < lcnt, cur,
                                                        zi16 + SENT)
            wbase = pl.multiple_of(lbase + ((cnt >> 3) << 3), 8)
            pltpu.sync_copy(chunkbuf.at[pl.ds(0, 584)],
                            lists.at[pl.ds(wbase, 584)])
            newcnt = cnt + lcnt - base
            clen = newcnt & 7
            cidx = jnp.clip(lcnt - clen + i16, 0, 591)
            cvals = plsc.load_gather(chunkbuf, [cidx])
            plsc.store_scatter(chunkbuf, [i16], cvals, mask=i16 < clen)
            return newcnt
        cnt = lax.fori_loop(0, EPT // PCH, prep_body, jnp.int32(0))
        for g in range(5):
            chunkbuf[pl.ds(16 * g, 16)] = zi16 + SENT
        endw = pl.multiple_of(lbase + (((cnt + 7) >> 3) << 3), 8)
        pltpu.sync_copy(chunkbuf.at[pl.ds(0, 64)], lists.at[pl.ds(endw, 64)])
        pltpu.sync_copy(chunkbuf.at[pl.ds(0, 64)],
                        lists.at[pl.ds(endw + 64, 64)])
        nchunks = (cnt + CHUNK - 1) // CHUNK

        def head_body(h, carry0):
            def zrow(r, carry):
                for c in range(FE // 16):
                    staging[0][r, pl.ds(16 * c, 16)] = z16
                return carry
            lax.fori_loop(0, CHUNK, zrow, 0)
            offs = s * RPT
            def zacc(z, carry):
                pltpu.sync_copy(staging[0],
                                acc.at[pl.ds(offs + z * CHUNK, CHUNK)])
                return carry
            lax.fori_loop(0, RPT // CHUNK, zacc, 0)
            plsc.subcore_barrier()

            hN = h * N

            def stage_issue(k, b):
                pltpu.sync_copy(
                    lists.at[pl.ds(pl.multiple_of(lbase + k * CHUNK, 8), CHUNK)],
                    srcbuf[b])
                for g in range(CHUNK // 16):
                    v16 = srcbuf[b][pl.ds(16 * g, 16)]
                    loc16 = v16 & (2 ** 13 - 1)
                    s16 = v16 >> 13
                    ok = loc16 != SENT
                    okbuf[b][pl.ds(16 * g, 16)] = jnp.where(ok, 1.0, 0.0)
                    locbuf[b][pl.ds(16 * g, 16)] = loc16
                    sidxbuf[b][pl.ds(16 * g, 16)] = s16 + hN
                    dstbuf[b][pl.ds(16 * g, 16)] = loc16 + lo + hN
                pltpu.async_copy(xwe_h.at[sidxbuf[b]], staging[b], sem[b])
                pltpu.async_copy(asr_h.at[sidxbuf[b]], sabuf[b], sem2[b])
                pltpu.async_copy(adr_h.at[dstbuf[b]], dabuf[b], sem3[b])

            def wait_gathers(b):
                pltpu.make_async_copy(xwe_h.at[sidxbuf[b]], staging[b], sem[b]).wait()
                pltpu.make_async_copy(asr_h.at[sidxbuf[b]], sabuf[b], sem2[b]).wait()
                pltpu.make_async_copy(adr_h.at[dstbuf[b]], dabuf[b], sem3[b]).wait()

            def scale_scatter(b):
                def srow(r, carry2):
                    lg = sabuf[b][r, pl.ds(0, 16)] + dabuf[b][r, pl.ds(0, 16)]
                    lg = jnp.where(lg >= 0, lg, 0.2 * lg)
                    okspl = plsc.load_gather(
                        okbuf[b], [jnp.zeros((16,), jnp.int32) + r])
                    wspl = jnp.exp(lg) * okspl
                    for c in range(FE // 16):
                        staging[b][r, pl.ds(16 * c, 16)] = (
                            staging[b][r, pl.ds(16 * c, 16)] * wspl)
                    return carry2
                lax.fori_loop(0, CHUNK, srow, 0)
                pltpu.sync_copy(staging[b], acc.at[locbuf[b]], add=True)

            stage_issue(0, 0)

            def pair_body(j, carry):
                k0 = 2 * j
                stage_issue(jnp.minimum(k0 + 1, nchunks), 1)
                wait_gathers(0)
                scale_scatter(0)
                stage_issue(jnp.minimum(k0 + 2, nchunks), 0)
                wait_gathers(1)
                scale_scatter(1)
                return carry
            lax.fori_loop(0, (nchunks + 1) // 2, pair_body, 0)
            wait_gathers(0)
            plsc.subcore_barrier()

            obase = h * 2 * NACC + sc * NACC + offs
            def cpout(z, carry):
                pltpu.sync_copy(acc.at[pl.ds(offs + z * CHUNK, CHUNK)],
                                out.at[pl.ds(obase + z * CHUNK, CHUNK)])
                return carry
            lax.fori_loop(0, RPT // CHUNK, cpout, 0)
            plsc.subcore_barrier()
            return carry0
        lax.fori_loop(0, 2, head_body, 0)

    return body(srcp, dstp, asrep, adrep, xwef)


def _mid_body(s0_ref, s1_ref, den_ref, bias_ref, h_ref, ps_ref, pq_ref):
    den0 = den_ref[:, 0:1]
    den1 = den_ref[:, 1:2]
    h = (s0_ref[:] / (den0 + 1e-16) + s1_ref[:] / (den1 + 1e-16)) * 0.5
    h = h + bias_ref[:]
    h_ref[:] = h
    ps_ref[0, 0, :] = jnp.sum(h, axis=0)
    pq_ref[0, 0, :] = jnp.sum(h * h, axis=0)


def _mid(s0, s1, den, bias):
    return pl.pallas_call(
        _mid_body,
        grid=(N_BLKS,),
        in_specs=[
            pl.BlockSpec((ROW_BLK, HID), lambda i: (i, 0)),
            pl.BlockSpec((ROW_BLK, HID), lambda i: (i, 0)),
            pl.BlockSpec((ROW_BLK, 2), lambda i: (i, 0)),
            pl.BlockSpec((1, HID), lambda i: (0, 0)),
        ],
        out_specs=[
            pl.BlockSpec((ROW_BLK, HID), lambda i: (i, 0)),
            pl.BlockSpec((1, 1, HID), lambda i: (i, 0, 0)),
            pl.BlockSpec((1, 1, HID), lambda i: (i, 0, 0)),
        ],
        out_shape=[
            jax.ShapeDtypeStruct((N, HID), jnp.float32),
            jax.ShapeDtypeStruct((N_BLKS, 1, HID), jnp.float32),
            jax.ShapeDtypeStruct((N_BLKS, 1, HID), jnp.float32),
        ],
    )(s0, s1, den, bias)


def _mlp_body(h_ref, sc_ref, sh_ref, w1_ref, b1_ref, w2_ref, b2_ref,
              w3_ref, b3_ref, w4_ref, b4_ref, o_ref):
    h = _lrelu(h_ref[:] * sc_ref[:] + sh_ref[:])
    h = _lrelu(jnp.dot(h, w1_ref[:], preferred_element_type=jnp.float32) + b1_ref[:])
    h = _lrelu(jnp.dot(h, w2_ref[:], preferred_element_type=jnp.float32) + b2_ref[:])
    h = _lrelu(jnp.dot(h, w3_ref[:], preferred_element_type=jnp.float32) + b3_ref[:])
    h = _lrelu(jnp.dot(h, w4_ref[:], preferred_element_type=jnp.float32) + b4_ref[:])
    o_ref[:] = h


def _mlp(h, scale, shift, w1, b1, w2, b2, w3, b3, w4, b4):
    full = lambda shape: pl.BlockSpec(shape, lambda i: tuple(0 for _ in shape))
    return pl.pallas_call(
        _mlp_body,
        grid=(N_BLKS,),
        in_specs=[
            pl.BlockSpec((ROW_BLK, HID), lambda i: (i, 0)),
            full((1, HID)), full((1, HID)),
            full((HID, HID // 2)), full((1, HID // 2)),
            full((HID // 2, HID // 4)), full((1, HID // 4)),
            full((HID // 4, HID // 4)), full((1, HID // 4)),
            full((HID // 4, 128)), full((1, 128)),
        ],
        out_specs=pl.BlockSpec((ROW_BLK, 128), lambda i: (i, 0)),
        out_shape=jax.ShapeDtypeStruct((N, 128), jnp.float32),
    )(h, scale, shift, w1, b1, w2, b2, w3, b3, w4, b4)


def kernel(normal_features, noise, edge_index, batch, W, att_src, att_dst,
           conv_bias, bn_gamma, bn_beta, w1, b1, w2, b2, w3, b3, w4, b4):
    x = jnp.concatenate([normal_features, noise], axis=1)
    loop = jnp.arange(N, dtype=edge_index.dtype)
    src = jnp.concatenate([edge_index[0], loop])
    dst = jnp.concatenate([edge_index[1], loop])

    z = jnp.zeros((HID,), jnp.float32)
    att_mat = jnp.stack([
        jnp.concatenate([att_src[0], z]),
        jnp.concatenate([z, att_src[1]]),
        jnp.concatenate([att_dst[0], z]),
        jnp.concatenate([z, att_dst[1]]),
    ], axis=1)

    xw0e, xw1e, a = _front(x, W, att_mat)

    m = _lrelu(jnp.max(a[:, 0:2], axis=0) + jnp.max(a[:, 2:4], axis=0))
    xwef = jnp.concatenate([xw0e * jnp.exp(-m[0]), xw1e * jnp.exp(-m[1])])

    pad = jnp.zeros((EL_PAD - EL,), dtype=edge_index.dtype)
    srcp = jnp.concatenate([src, pad])
    dstp = jnp.concatenate([dst, pad])

    asrep = jnp.broadcast_to(a[:, 0:2].T.reshape(2 * N, 1), (2 * N, 16))
    adrep = jnp.broadcast_to(a[:, 2:4].T.reshape(2 * N, 1), (2 * N, 16))
    outf = _edge_sc(srcp, dstp, asrep, adrep, xwef)
    s0e = jnp.concatenate([outf[:NSC], outf[NACC:NACC + NSC]], axis=0)
    s1e = jnp.concatenate([outf[2 * NACC:2 * NACC + NSC],
                           outf[3 * NACC:3 * NACC + NSC]], axis=0)
    s0 = s0e[:, :HID]
    s1 = s1e[:, :HID]
    den = jnp.stack([s0e[:, HID], s1e[:, HID]], axis=1)

    h, ps, pq = _mid(s0, s1, den, conv_bias.reshape(1, HID))
    mu = ps.sum(axis=(0, 1)) / N
    var = pq.sum(axis=(0, 1)) / N - mu * mu
    scale = bn_gamma / jnp.sqrt(var + 1e-5)
    shift = bn_beta - mu * scale

    return _mlp(h, scale.reshape(1, HID), shift.reshape(1, HID),
                w1, b1.reshape(1, -1), w2, b2.reshape(1, -1),
                w3, b3.reshape(1, -1), w4, b4.reshape(1, -1))


# speedup vs baseline: 2.5675x; 1.0006x over previous
import functools

import jax
import jax.numpy as jnp
from jax import lax
from jax.experimental import pallas as pl
from jax.experimental.pallas import tpu as pltpu
from jax.experimental.pallas import tpu_sc as plsc

N = 10000
E = 320000
F_IN = 160
H = 2
HID = 256

ROW_BLK = 2000
N_BLKS = N // ROW_BLK

LROW = 21504
EL = E + N
CHUNK = 64
CPT = 324
EPT = CPT * CHUNK
EL_PAD = 16 * EPT
NSC = 5000
NACC = 5120
RPT = NACC // 16
FE = HID + 16


def _lrelu(v):
    return jnp.where(v >= 0, v, 0.2 * v)


def _front_body(x_ref, w_ref, att_ref, xw0_ref, xw1_ref, a_ref):
    xw = jnp.dot(x_ref[:], w_ref[:], preferred_element_type=jnp.float32)
    one = jnp.ones((ROW_BLK, 1), jnp.float32)
    zpad = jnp.zeros((ROW_BLK, FE - HID - 1), jnp.float32)
    xw0_ref[:] = jnp.concatenate([xw[:, :HID], one, zpad], axis=1)
    xw1_ref[:] = jnp.concatenate([xw[:, HID:], one, zpad], axis=1)
    a_ref[:] = jnp.dot(xw, att_ref[:], preferred_element_type=jnp.float32)


def _front(x, W, att_mat):
    return pl.pallas_call(
        _front_body,
        grid=(N_BLKS,),
        in_specs=[
            pl.BlockSpec((ROW_BLK, F_IN), lambda i: (i, 0)),
            pl.BlockSpec((F_IN, H * HID), lambda i: (0, 0)),
            pl.BlockSpec((H * HID, 4), lambda i: (0, 0)),
        ],
        out_specs=[
            pl.BlockSpec((ROW_BLK, FE), lambda i: (i, 0)),
            pl.BlockSpec((ROW_BLK, FE), lambda i: (i, 0)),
            pl.BlockSpec((ROW_BLK, 4), lambda i: (i, 0)),
        ],
        out_shape=[
            jax.ShapeDtypeStruct((N, FE), jnp.float32),
            jax.ShapeDtypeStruct((N, FE), jnp.float32),
            jax.ShapeDtypeStruct((N, 4), jnp.float32),
        ],
    )(x, W, att_mat)


def _edge_sc(srcp, dstp, asrep, adrep, xwef):
    mesh = plsc.VectorSubcoreMesh(core_axis_name="c", subcore_axis_name="s")

    @functools.partial(
        pl.kernel, mesh=mesh,
        compiler_params=pltpu.CompilerParams(needs_layout_passes=False,
                                             use_tc_tiling_on_sc=False),
        out_type=jax.ShapeDtypeStruct((4 * NACC, FE), jnp.float32),
        scratch_types=[
            [pltpu.VMEM((CHUNK,), jnp.int32)] * 2,
            [pltpu.VMEM((CHUNK,), jnp.int32)] * 2,
            [pltpu.VMEM((CHUNK,), jnp.int32)] * 2,
            [pltpu.VMEM((CHUNK,), jnp.int32)] * 2,
            [pltpu.VMEM((CHUNK,), jnp.float32)] * 2,
            [pltpu.VMEM((CHUNK, 16), jnp.float32)] * 2,
            [pltpu.VMEM((CHUNK, 16), jnp.float32)] * 2,
            [pltpu.VMEM((CHUNK, FE), jnp.float32)] * 2,
            pltpu.VMEM((1168,), jnp.int32),
            pltpu.VMEM((576,), jnp.int32),
            pltpu.VMEM((576,), jnp.int32),
            pltpu.HBM((2 * 16 * LROW,), jnp.int32),
            pltpu.VMEM_SHARED((NACC, FE), jnp.float32),
            [pltpu.SemaphoreType.DMA] * 2,
            [pltpu.SemaphoreType.DMA] * 2,
            [pltpu.SemaphoreType.DMA] * 2,
        ],
    )
    def body(srcp_h, dstp_h, asr_h, adr_h, xwe_h, out,
             srcbuf, dstbuf, sidxbuf, locbuf, okbuf, sabuf, dabuf, staging,
             chunkbuf, pbuf_s, pbuf_d, lists, acc, sem, sem2, sem3):
        sc = lax.axis_index("c")
        s = lax.axis_index("s")
        tile_base = s * EPT
        lo = sc * NSC
        lbase = (sc * 16 + s) * LROW

        z16 = jnp.zeros((16,), jnp.float32)
        zi16 = jnp.zeros((16,), jnp.int32)
        i16 = lax.iota(jnp.int32, 16)
        SENT = NACC - 1

        PCH = 576
        def prep_body(k, carry):
            cnt8, f = carry
            off = tile_base + k * PCH
            pltpu.sync_copy(srcp_h.at[pl.ds(off, PCH)], pbuf_s)
            pltpu.sync_copy(dstp_h.at[pl.ds(off, PCH)], pbuf_d)
            lcnt = f
            for g in range(PCH // 16):
                s16 = pbuf_s[pl.ds(16 * g, 16)]
                d16 = pbuf_d[pl.ds(16 * g, 16)]
                eid = off + 16 * g + i16
                m = (eid < EL) & (d16 >= lo) & (d16 < lo + NSC)
                mi = jnp.where(m, 1, 0).astype(jnp.int32)
                pos = lcnt + plsc.cumsum(mi) - mi
                plsc.store_scatter(chunkbuf, [pos],
                                   (s16 << 13) | (d16 - lo), mask=m)
                lcnt = lcnt + jnp.sum(mi)
            flushed = lcnt >= PCH
            @pl.when(flushed)
            def _flush():
                pltpu.sync_copy(
                    chunkbuf.at[pl.ds(0, PCH)],
                    lists.at[pl.ds(pl.multiple_of(lbase + cnt8, 8), PCH)])
                for g in range(PCH // 16):
                    chunkbuf[pl.ds(16 * g, 16)] = (
                        chunkbuf[pl.ds(PCH + 16 * g, 16)])
            cnt8 = jnp.where(flushed, cnt8 + PCH, cnt8)
            f = jnp.where(flushed, lcnt - PCH, lcnt)
            return (cnt8, f)
        cnt8, f = lax.fori_loop(0, EPT // PCH, prep_body,
                                (jnp.int32(0), jnp.int32(0)))
        for g in range(37):
            idx = 16 * g + i16
            cur = chunkbuf[pl.ds(16 * g, 16)]
            chunkbuf[pl.ds(16 * g, 16)] = jnp.where(idx < f, cur, zi16 + SENT)
        tb = pl.multiple_of(lbase + cnt8, 8)
        pltpu.sync_copy(chunkbuf.at[pl.ds(0, 584)], lists.at[pl.ds(tb, 584)])
        for g in range(5):
            chunkbuf[pl.ds(16 * g, 16)] = zi16 + SENT
        pltpu.sync_copy(chunkbuf.at[pl.ds(0, 64)],
                        lists.at[pl.ds(tb + 584, 64)])
        pltpu.sync_copy(chunkbuf.at[pl.ds(0, 64)],
                        lists.at[pl.ds(tb + 648, 64)])
        cnt = cnt8 + f
        nchunks = (cnt + CHUNK - 1) // CHUNK

        def head_body(h, carry0):
            def zrow(r, carry):
                for c in range(FE // 16):
                    staging[0][r, pl.ds(16 * c, 16)] = z16
                return carry
            lax.fori_loop(0, CHUNK, zrow, 0)
            offs = s * RPT
            def zacc(z, carry):
                pltpu.sync_copy(staging[0],
                                acc.at[pl.ds(offs + z * CHUNK, CHUNK)])
                return carry
            lax.fori_loop(0, RPT // CHUNK, zacc, 0)
            plsc.subcore_barrier()

            hN = h * N

            def stage_issue(k, b):
                pltpu.sync_copy(
                    lists.at[pl.ds(pl.multiple_of(lbase + k * CHUNK, 8), CHUNK)],
                    srcbuf[b])
                for g in range(CHUNK // 16):
                    v16 = srcbuf[b][pl.ds(16 * g, 16)]
                    loc16 = v16 & (2 ** 13 - 1)
                    s16 = v16 >> 13
                    ok = loc16 != SENT
                    okbuf[b][pl.ds(16 * g, 16)] = jnp.where(ok, 1.0, 0.0)
                    locbuf[b][pl.ds(16 * g, 16)] = loc16
                    sidxbuf[b][pl.ds(16 * g, 16)] = s16 + hN
                    dstbuf[b][pl.ds(16 * g, 16)] = loc16 + lo + hN
                pltpu.async_copy(xwe_h.at[sidxbuf[b]], staging[b], sem[b])
                pltpu.async_copy(asr_h.at[sidxbuf[b]], sabuf[b], sem2[b])
                pltpu.async_copy(adr_h.at[dstbuf[b]], dabuf[b], sem3[b])

            def wait_gathers(b):
                pltpu.make_async_copy(xwe_h.at[sidxbuf[b]], staging[b], sem[b]).wait()
                pltpu.make_async_copy(asr_h.at[sidxbuf[b]], sabuf[b], sem2[b]).wait()
                pltpu.make_async_copy(adr_h.at[dstbuf[b]], dabuf[b], sem3[b]).wait()

            def scale_scatter(b):
                def srow(r, carry2):
                    lg = sabuf[b][r, pl.ds(0, 16)] + dabuf[b][r, pl.ds(0, 16)]
                    lg = jnp.where(lg >= 0, lg, 0.2 * lg)
                    okspl = plsc.load_gather(
                        okbuf[b], [jnp.zeros((16,), jnp.int32) + r])
                    wspl = jnp.exp(lg) * okspl
                    for c in range(FE // 16):
                        staging[b][r, pl.ds(16 * c, 16)] = (
                            staging[b][r, pl.ds(16 * c, 16)] * wspl)
                    return carry2
                lax.fori_loop(0, CHUNK, srow, 0)
                pltpu.sync_copy(staging[b], acc.at[locbuf[b]], add=True)

            stage_issue(0, 0)

            def pair_body(j, carry):
                k0 = 2 * j
                stage_issue(jnp.minimum(k0 + 1, nchunks), 1)
                wait_gathers(0)
                scale_scatter(0)
                stage_issue(jnp.minimum(k0 + 2, nchunks), 0)
                wait_gathers(1)
                scale_scatter(1)
                return carry
            lax.fori_loop(0, (nchunks + 1) // 2, pair_body, 0)
            wait_gathers(0)
            plsc.subcore_barrier()

            obase = h * 2 * NACC + sc * NACC + offs
            def cpout(z, carry):
                pltpu.sync_copy(acc.at[pl.ds(offs + z * CHUNK, CHUNK)],
                                out.at[pl.ds(obase + z * CHUNK, CHUNK)])
                return carry
            lax.fori_loop(0, RPT // CHUNK, cpout, 0)
            plsc.subcore_barrier()
            return carry0
        lax.fori_loop(0, 2, head_body, 0)

    return body(srcp, dstp, asrep, adrep, xwef)


def _mid_body(s0_ref, s1_ref, den_ref, bias_ref, h_ref, ps_ref, pq_ref):
    den0 = den_ref[:, 0:1]
    den1 = den_ref[:, 1:2]
    h = (s0_ref[:] / (den0 + 1e-16) + s1_ref[:] / (den1 + 1e-16)) * 0.5
    h = h + bias_ref[:]
    h_ref[:] = h
    ps_ref[0, 0, :] = jnp.sum(h, axis=0)
    pq_ref[0, 0, :] = jnp.sum(h * h, axis=0)


def _mid(s0, s1, den, bias):
    return pl.pallas_call(
        _mid_body,
        grid=(N_BLKS,),
        in_specs=[
            pl.BlockSpec((ROW_BLK, HID), lambda i: (i, 0)),
            pl.BlockSpec((ROW_BLK, HID), lambda i: (i, 0)),
            pl.BlockSpec((ROW_BLK, 2), lambda i: (i, 0)),
            pl.BlockSpec((1, HID), lambda i: (0, 0)),
        ],
        out_specs=[
            pl.BlockSpec((ROW_BLK, HID), lambda i: (i, 0)),
            pl.BlockSpec((1, 1, HID), lambda i: (i, 0, 0)),
            pl.BlockSpec((1, 1, HID), lambda i: (i, 0, 0)),
        ],
        out_shape=[
            jax.ShapeDtypeStruct((N, HID), jnp.float32),
            jax.ShapeDtypeStruct((N_BLKS, 1, HID), jnp.float32),
            jax.ShapeDtypeStruct((N_BLKS, 1, HID), jnp.float32),
        ],
    )(s0, s1, den, bias)


def _mlp_body(h_ref, sc_ref, sh_ref, w1_ref, b1_ref, w2_ref, b2_ref,
              w3_ref, b3_ref, w4_ref, b4_ref, o_ref):
    h = _lrelu(h_ref[:] * sc_ref[:] + sh_ref[:])
    h = _lrelu(jnp.dot(h, w1_ref[:], preferred_element_type=jnp.float32) + b1_ref[:])
    h = _lrelu(jnp.dot(h, w2_ref[:], preferred_element_type=jnp.float32) + b2_ref[:])
    h = _lrelu(jnp.dot(h, w3_ref[:], preferred_element_type=jnp.float32) + b3_ref[:])
    h = _lrelu(jnp.dot(h, w4_ref[:], preferred_element_type=jnp.float32) + b4_ref[:])
    o_ref[:] = h


def _mlp(h, scale, shift, w1, b1, w2, b2, w3, b3, w4, b4):
    full = lambda shape: pl.BlockSpec(shape, lambda i: tuple(0 for _ in shape))
    return pl.pallas_call(
        _mlp_body,
        grid=(N_BLKS,),
        in_specs=[
            pl.BlockSpec((ROW_BLK, HID), lambda i: (i, 0)),
            full((1, HID)), full((1, HID)),
            full((HID, HID // 2)), full((1, HID // 2)),
            full((HID // 2, HID // 4)), full((1, HID // 4)),
            full((HID // 4, HID // 4)), full((1, HID // 4)),
            full((HID // 4, 128)), full((1, 128)),
        ],
        out_specs=pl.BlockSpec((ROW_BLK, 128), lambda i: (i, 0)),
        out_shape=jax.ShapeDtypeStruct((N, 128), jnp.float32),
    )(h, scale, shift, w1, b1, w2, b2, w3, b3, w4, b4)


def kernel(normal_features, noise, edge_index, batch, W, att_src, att_dst,
           conv_bias, bn_gamma, bn_beta, w1, b1, w2, b2, w3, b3, w4, b4):
    x = jnp.concatenate([normal_features, noise], axis=1)
    loop = jnp.arange(N, dtype=edge_index.dtype)
    src = jnp.concatenate([edge_index[0], loop])
    dst = jnp.concatenate([edge_index[1], loop])

    z = jnp.zeros((HID,), jnp.float32)
    att_mat = jnp.stack([
        jnp.concatenate([att_src[0], z]),
        jnp.concatenate([z, att_src[1]]),
        jnp.concatenate([att_dst[0], z]),
        jnp.concatenate([z, att_dst[1]]),
    ], axis=1)

    xw0e, xw1e, a = _front(x, W, att_mat)

    m = _lrelu(jnp.max(a[:, 0:2], axis=0) + jnp.max(a[:, 2:4], axis=0))
    xwef = jnp.concatenate([xw0e * jnp.exp(-m[0]), xw1e * jnp.exp(-m[1])])

    pad = jnp.zeros((EL_PAD - EL,), dtype=edge_index.dtype)
    srcp = jnp.concatenate([src, pad])
    dstp = jnp.concatenate([dst, pad])

    asrep = jnp.broadcast_to(a[:, 0:2].T.reshape(2 * N, 1), (2 * N, 16))
    adrep = jnp.broadcast_to(a[:, 2:4].T.reshape(2 * N, 1), (2 * N, 16))
    outf = _edge_sc(srcp, dstp, asrep, adrep, xwef)
    s0e = jnp.concatenate([outf[:NSC], outf[NACC:NACC + NSC]], axis=0)
    s1e = jnp.concatenate([outf[2 * NACC:2 * NACC + NSC],
                           outf[3 * NACC:3 * NACC + NSC]], axis=0)
    s0 = s0e[:, :HID]
    s1 = s1e[:, :HID]
    den = jnp.stack([s0e[:, HID], s1e[:, HID]], axis=1)

    h, ps, pq = _mid(s0, s1, den, conv_bias.reshape(1, HID))
    mu = ps.sum(axis=(0, 1)) / N
    var = pq.sum(axis=(0, 1)) / N - mu * mu
    scale = bn_gamma / jnp.sqrt(var + 1e-5)
    shift = bn_beta - mu * scale

    return _mlp(h, scale.reshape(1, HID), shift.reshape(1, HID),
                w1, b1.reshape(1, -1), w2, b2.reshape(1, -1),
                w3, b3.reshape(1, -1), w4, b4.reshape(1, -1))
